# baseline (device time: 18694 ns/iter reference)
import jax
import jax.numpy as jnp
from jax import lax
from jax.experimental import pallas as pl
from jax.experimental.pallas import tpu as pltpu

N_DEV = 4
B, Sq, Skv = 2, 256, 256
HQ_GLOBAL, Dh = 16, 64
H_PER = HQ_GLOBAL // N_DEV
D_MODEL = 512
BLOCK = 64
R = (B * Sq) // N_DEV
HALF = D_MODEL // 2


def kernel(x, Wq, K_ext, V_ext, Wo):
    my_pos = lax.axis_index("i")
    K = lax.dynamic_slice_in_dim(K_ext, my_pos * H_PER, H_PER, axis=2)
    V = lax.dynamic_slice_in_dim(V_ext, my_pos * H_PER, H_PER, axis=2)
    K = jnp.transpose(K, (2, 3, 0, 1)).reshape(H_PER, Dh, B * Skv)
    V = jnp.transpose(V, (2, 0, 1, 3)).reshape(H_PER, B * Skv, Dh)
    K = K.astype(jnp.bfloat16)
    V = V.astype(jnp.bfloat16)
    x_flat = x.reshape(B * Sq, D_MODEL).astype(jnp.bfloat16)
    Wq = Wq.astype(jnp.bfloat16)
    Wo = Wo.astype(jnp.bfloat16)

    def body(x_ref, wq_ref, k_ref, v_ref, wo_ref, out_ref,
             part_ref, rs_ref, ag_ref,
             rs_send_sems, rs_recv_sems, ag_send_sems, ag_recv_sems):
        my = lax.axis_index("i")

        barrier_sem = pltpu.get_barrier_semaphore()
        for d in range(1, N_DEV):
            pl.semaphore_signal(
                barrier_sem, inc=1,
                device_id=(lax.rem(my + d, N_DEV),),
                device_id_type=pl.DeviceIdType.MESH,
            )
        pl.semaphore_wait(barrier_sem, N_DEV - 1)

        qb = lax.broadcasted_iota(jnp.int32, (R, R), 0) // BLOCK
        kb = lax.broadcasted_iota(jnp.int32, (R, R), 1) // BLOCK
        mask = qb == kb

        def compute_chunk(c):
            rows = pl.ds(c * R, R)
            q_all = jnp.dot(x_ref[rows, :], wq_ref[...],
                            preferred_element_type=jnp.float32
                            ).astype(jnp.bfloat16)
            ctx_parts = []
            for h in range(H_PER):
                q_bh = q_all[:, h * Dh:(h + 1) * Dh]
                k_bh = k_ref[h, :, rows]
                v_bh = v_ref[h, rows, :]
                s = jnp.dot(q_bh, k_bh,
                            preferred_element_type=jnp.float32) * 0.125
                w = jnp.exp(jnp.where(mask, s, -1e9))
                w = w / jnp.sum(w, axis=-1, keepdims=True)
                ctx_parts.append(jnp.dot(w.astype(jnp.bfloat16), v_bh,
                                         preferred_element_type=jnp.float32
                                         ).astype(jnp.bfloat16))
            ctx = jnp.concatenate(ctx_parts, axis=1)
            return jnp.dot(ctx, wo_ref[...],
                           preferred_element_type=jnp.float32)

        def rs_desc(hf, j, dev):
            return pltpu.make_async_remote_copy(
                src_ref=part_ref.at[hf, j],
                dst_ref=rs_ref.at[hf, j],
                send_sem=rs_send_sems.at[hf * 3 + j],
                recv_sem=rs_recv_sems.at[hf * 3 + j],
                device_id=(dev,),
                device_id_type=pl.DeviceIdType.MESH,
            )

        def ag_desc(hf, d):
            return pltpu.make_async_remote_copy(
                src_ref=ag_ref.at[hf, my],
                dst_ref=ag_ref.at[hf, my],
                send_sem=ag_send_sems.at[hf * 3 + d - 1],
                recv_sem=ag_recv_sems.at[hf * 4 + my],
                device_id=(lax.rem(my + d, N_DEV),),
                device_id_type=pl.DeviceIdType.MESH,
            )

        def ag_recv_desc(hf, src):
            return pltpu.make_async_remote_copy(
                src_ref=ag_ref.at[hf, 0],
                dst_ref=ag_ref.at[hf, src],
                send_sem=rs_send_sems.at[0],
                recv_sem=ag_recv_sems.at[hf * 4 + src],
                device_id=(src,),
                device_id_type=pl.DeviceIdType.MESH,
            )

        for j in range(N_DEV - 1):
            tgt = lax.rem(my + 1 + j, N_DEV)
            chunk = compute_chunk(tgt).astype(jnp.bfloat16)
            for hf in range(2):
                part_ref[hf, j] = chunk[:, hf * HALF:(hf + 1) * HALF]
                rs_desc(hf, j, tgt).start()
        own = compute_chunk(my)

        for hf in range(2):
            for j in range(N_DEV - 1):
                rs_desc(hf, j, my).wait_recv()
            red = (own[:, hf * HALF:(hf + 1) * HALF]
                   + rs_ref[hf, 0].astype(jnp.float32)
                   + rs_ref[hf, 1].astype(jnp.float32)
                   + rs_ref[hf, 2].astype(jnp.float32))
            for c in range(N_DEV):
                @pl.when(c == my)
                def _():
                    ag_ref[hf, c] = red.astype(jnp.bfloat16)
            for d in range(1, N_DEV):
                ag_desc(hf, d).start()

        for hf in range(2):
            for d in range(1, N_DEV):
                ag_recv_desc(hf, lax.rem(my + d, N_DEV)).wait_recv()
        for c in range(N_DEV):
            for hf in range(2):
                out_ref[c * R:(c + 1) * R, hf * HALF:(hf + 1) * HALF] = (
                    ag_ref[hf, c].astype(jnp.float32))

        for j in range(N_DEV - 1):
            for hf in range(2):
                rs_desc(hf, j, lax.rem(my + 1 + j, N_DEV)).wait_send()
        for hf in range(2):
            for d in range(1, N_DEV):
                ag_desc(hf, d).wait_send()

    out_shape = jax.ShapeDtypeStruct((B * Sq, D_MODEL), jnp.float32)
    out_flat = pl.pallas_call(
        body,
        out_shape=out_shape,
        in_specs=[pl.BlockSpec(memory_space=pltpu.VMEM)] * 5,
        out_specs=pl.BlockSpec(memory_space=pltpu.VMEM),
        scratch_shapes=[
            pltpu.VMEM((2, N_DEV - 1, R, HALF), jnp.bfloat16),
            pltpu.VMEM((2, N_DEV - 1, R, HALF), jnp.bfloat16),
            pltpu.VMEM((2, N_DEV, R, HALF), jnp.bfloat16),
            pltpu.SemaphoreType.DMA((2 * (N_DEV - 1),)),
            pltpu.SemaphoreType.DMA((2 * (N_DEV - 1),)),
            pltpu.SemaphoreType.DMA((2 * (N_DEV - 1),)),
            pltpu.SemaphoreType.DMA((2 * N_DEV,)),
        ],
        compiler_params=pltpu.CompilerParams(collective_id=0),
    )(x_flat, Wq, K, V, Wo)
    return out_flat.reshape(B, Sq, D_MODEL)


# device time: 16227 ns/iter; 1.1520x vs baseline; 1.1520x over previous
import jax
import jax.numpy as jnp
from jax import lax
from jax.experimental import pallas as pl
from jax.experimental.pallas import tpu as pltpu

N_DEV = 4
B, Sq, Skv = 2, 256, 256
HQ_GLOBAL, Dh = 16, 64
H_PER = HQ_GLOBAL // N_DEV
D_MODEL = 512
BLOCK = 64
R = (B * Sq) // N_DEV
HALF = D_MODEL // 2


def kernel(x, Wq, K_ext, V_ext, Wo):
    my_pos = lax.axis_index("i")
    K = lax.dynamic_slice_in_dim(K_ext, my_pos * H_PER, H_PER, axis=2)
    V = lax.dynamic_slice_in_dim(V_ext, my_pos * H_PER, H_PER, axis=2)
    K = jnp.transpose(K.astype(jnp.bfloat16), (2, 3, 0, 1)).reshape(
        H_PER, Dh, B * Skv)
    V = jnp.transpose(V.astype(jnp.bfloat16), (2, 0, 1, 3)).reshape(
        H_PER, B * Skv, Dh)
    x_flat = x.reshape(B * Sq, D_MODEL)

    def body(x_ref, wq_ref, k_ref, v_ref, wo_ref, out_ref,
             wq_bf, wo_bf,
             part_ref, rs_ref, ag_ref,
             rs_send_sems, rs_recv_sems, ag_send_sems, ag_recv_sems):
        my = lax.axis_index("i")

        barrier_sem = pltpu.get_barrier_semaphore()
        for d in range(1, N_DEV):
            pl.semaphore_signal(
                barrier_sem, inc=1,
                device_id=(lax.rem(my + d, N_DEV),),
                device_id_type=pl.DeviceIdType.MESH,
            )

        wq_bf[...] = wq_ref[...].astype(jnp.bfloat16)
        wo_bf[...] = wo_ref[...].astype(jnp.bfloat16)

        qb = lax.broadcasted_iota(jnp.int32, (R, R), 0) // BLOCK
        kb = lax.broadcasted_iota(jnp.int32, (R, R), 1) // BLOCK
        mask = qb == kb

        def compute_chunk(c):
            rows = pl.ds(c * R, R)
            q_all = jnp.dot(x_ref[rows, :].astype(jnp.bfloat16), wq_bf[...],
                            preferred_element_type=jnp.float32
                            ).astype(jnp.bfloat16)
            ctx_parts = []
            for h in range(H_PER):
                q_bh = q_all[:, h * Dh:(h + 1) * Dh]
                k_bh = k_ref[h, :, rows]
                v_bh = v_ref[h, rows, :]
                s = jnp.dot(q_bh, k_bh,
                            preferred_element_type=jnp.float32) * 0.125
                w = jnp.exp(jnp.where(mask, s, -1e9))
                w = w / jnp.sum(w, axis=-1, keepdims=True)
                ctx_parts.append(jnp.dot(w.astype(jnp.bfloat16), v_bh,
                                         preferred_element_type=jnp.float32
                                         ).astype(jnp.bfloat16))
            ctx = jnp.concatenate(ctx_parts, axis=1)
            return jnp.dot(ctx, wo_bf[...],
                           preferred_element_type=jnp.float32)

        def rs_desc(hf, j, dev):
            return pltpu.make_async_remote_copy(
                src_ref=part_ref.at[hf, j],
                dst_ref=rs_ref.at[hf, j],
                send_sem=rs_send_sems.at[hf * 3 + j],
                recv_sem=rs_recv_sems.at[hf * 3 + j],
                device_id=(dev,),
                device_id_type=pl.DeviceIdType.MESH,
            )

        def ag_desc(hf, d):
            return pltpu.make_async_remote_copy(
                src_ref=ag_ref.at[hf, my],
                dst_ref=ag_ref.at[hf, my],
                send_sem=ag_send_sems.at[hf * 3 + d - 1],
                recv_sem=ag_recv_sems.at[hf * 4 + my],
                device_id=(lax.rem(my + d, N_DEV),),
                device_id_type=pl.DeviceIdType.MESH,
            )

        def ag_recv_desc(hf, src):
            return pltpu.make_async_remote_copy(
                src_ref=ag_ref.at[hf, 0],
                dst_ref=ag_ref.at[hf, src],
                send_sem=rs_send_sems.at[0],
                recv_sem=ag_recv_sems.at[hf * 4 + src],
                device_id=(src,),
                device_id_type=pl.DeviceIdType.MESH,
            )

        for j in range(N_DEV - 1):
            tgt = lax.rem(my + 1 + j, N_DEV)
            chunk = compute_chunk(tgt).astype(jnp.bfloat16)
            for hf in range(2):
                part_ref[hf, j] = chunk[:, hf * HALF:(hf + 1) * HALF]
            if j == 0:
                pl.semaphore_wait(barrier_sem, N_DEV - 1)
            for hf in range(2):
                rs_desc(hf, j, tgt).start()
        own = compute_chunk(my)

        for hf in range(2):
            for j in range(N_DEV - 1):
                rs_desc(hf, j, my).wait_recv()
            red = (own[:, hf * HALF:(hf + 1) * HALF]
                   + rs_ref[hf, 0].astype(jnp.float32)
                   + rs_ref[hf, 1].astype(jnp.float32)
                   + rs_ref[hf, 2].astype(jnp.float32))
            for c in range(N_DEV):
                @pl.when(c == my)
                def _():
                    ag_ref[hf, c] = red.astype(jnp.bfloat16)
            for d in range(1, N_DEV):
                ag_desc(hf, d).start()

        for hf in range(2):
            for d in range(1, N_DEV):
                ag_recv_desc(hf, lax.rem(my + d, N_DEV)).wait_recv()
        for c in range(N_DEV):
            for hf in range(2):
                out_ref[c * R:(c + 1) * R, hf * HALF:(hf + 1) * HALF] = (
                    ag_ref[hf, c].astype(jnp.float32))

        for j in range(N_DEV - 1):
            for hf in range(2):
                rs_desc(hf, j, lax.rem(my + 1 + j, N_DEV)).wait_send()
        for hf in range(2):
            for d in range(1, N_DEV):
                ag_desc(hf, d).wait_send()

    out_shape = jax.ShapeDtypeStruct((B * Sq, D_MODEL), jnp.float32)
    out_flat = pl.pallas_call(
        body,
        out_shape=out_shape,
        in_specs=[pl.BlockSpec(memory_space=pltpu.VMEM)] * 5,
        out_specs=pl.BlockSpec(memory_space=pltpu.VMEM),
        scratch_shapes=[
            pltpu.VMEM((D_MODEL, H_PER * Dh), jnp.bfloat16),
            pltpu.VMEM((H_PER * Dh, D_MODEL), jnp.bfloat16),
            pltpu.VMEM((2, N_DEV - 1, R, HALF), jnp.bfloat16),
            pltpu.VMEM((2, N_DEV - 1, R, HALF), jnp.bfloat16),
            pltpu.VMEM((2, N_DEV, R, HALF), jnp.bfloat16),
            pltpu.SemaphoreType.DMA((2 * (N_DEV - 1),)),
            pltpu.SemaphoreType.DMA((2 * (N_DEV - 1),)),
            pltpu.SemaphoreType.DMA((2 * (N_DEV - 1),)),
            pltpu.SemaphoreType.DMA((2 * N_DEV,)),
        ],
        compiler_params=pltpu.CompilerParams(collective_id=0),
    )(x_flat, Wq, K, V, Wo)
    return out_flat.reshape(B, Sq, D_MODEL)


# device time: 14931 ns/iter; 1.2520x vs baseline; 1.0868x over previous
import jax
import jax.numpy as jnp
from jax import lax
from jax.experimental import pallas as pl
from jax.experimental.pallas import tpu as pltpu

N_DEV = 4
B, Sq, Skv = 2, 256, 256
HQ_GLOBAL, Dh = 16, 64
H_PER = HQ_GLOBAL // N_DEV
D_MODEL = 512
BLOCK = 64
R = (B * Sq) // N_DEV
NHALF = 4
HALF = D_MODEL // NHALF


def kernel(x, Wq, K_ext, V_ext, Wo):
    my_pos = lax.axis_index("i")
    K = lax.dynamic_slice_in_dim(K_ext, my_pos * H_PER, H_PER, axis=2)
    V = lax.dynamic_slice_in_dim(V_ext, my_pos * H_PER, H_PER, axis=2)
    K = jnp.transpose(K.astype(jnp.bfloat16), (2, 3, 0, 1)).reshape(
        H_PER, Dh, B * Skv)
    V = jnp.transpose(V.astype(jnp.bfloat16), (2, 0, 1, 3)).reshape(
        H_PER, B * Skv, Dh)
    x_flat = x.reshape(B * Sq, D_MODEL)

    def body(x_ref, wq_ref, k_ref, v_ref, wo_ref, out_ref,
             wq_bf, wo_bf,
             part_ref, rs_ref, ag_ref,
             rs_send_sems, rs_recv_sems, ag_send_sems, ag_recv_sems):
        my = lax.axis_index("i")

        barrier_sem = pltpu.get_barrier_semaphore()
        for d in range(1, N_DEV):
            pl.semaphore_signal(
                barrier_sem, inc=1,
                device_id=(lax.rem(my + d, N_DEV),),
                device_id_type=pl.DeviceIdType.MESH,
            )

        wq_bf[...] = (wq_ref[...] * 0.125).astype(jnp.bfloat16)
        wo_bf[...] = wo_ref[...].astype(jnp.bfloat16)

        qb = lax.broadcasted_iota(jnp.int32, (R, R), 0) // BLOCK
        kb = lax.broadcasted_iota(jnp.int32, (R, R), 1) // BLOCK
        mask = qb == kb

        def compute_ctx(c):
            rows = pl.ds(c * R, R)
            q_all = jnp.dot(x_ref[rows, :].astype(jnp.bfloat16), wq_bf[...],
                            preferred_element_type=jnp.float32
                            ).astype(jnp.bfloat16)
            ctx_parts = []
            for h in range(H_PER):
                q_bh = q_all[:, h * Dh:(h + 1) * Dh]
                k_bh = k_ref[h, :, rows]
                v_bh = v_ref[h, rows, :]
                s = jnp.dot(q_bh, k_bh,
                            preferred_element_type=jnp.float32)
                w = jnp.exp(jnp.where(mask, s, -1e9))
                recip = 1.0 / jnp.sum(w, axis=-1, keepdims=True)
                ctx = jnp.dot(w.astype(jnp.bfloat16), v_bh,
                              preferred_element_type=jnp.float32)
                ctx_parts.append((ctx * recip).astype(jnp.bfloat16))
            return ctx_parts

        def project(ctx_parts, wo_cols):
            acc = jnp.dot(ctx_parts[0], wo_cols[0 * Dh:1 * Dh],
                          preferred_element_type=jnp.float32)
            for h in range(1, H_PER):
                acc = acc + jnp.dot(ctx_parts[h],
                                    wo_cols[h * Dh:(h + 1) * Dh],
                                    preferred_element_type=jnp.float32)
            return acc

        def compute_chunk(c):
            return project(compute_ctx(c), wo_bf[...])

        def rs_desc(hf, j, dev):
            return pltpu.make_async_remote_copy(
                src_ref=part_ref.at[hf, j],
                dst_ref=rs_ref.at[hf, j],
                send_sem=rs_send_sems.at[hf * 3 + j],
                recv_sem=rs_recv_sems.at[hf * 3 + j],
                device_id=(dev,),
                device_id_type=pl.DeviceIdType.MESH,
            )

        def ag_desc(hf, d):
            return pltpu.make_async_remote_copy(
                src_ref=ag_ref.at[hf, my],
                dst_ref=ag_ref.at[hf, my],
                send_sem=ag_send_sems.at[hf * 3 + d - 1],
                recv_sem=ag_recv_sems.at[hf * 4 + my],
                device_id=(lax.rem(my + d, N_DEV),),
                device_id_type=pl.DeviceIdType.MESH,
            )

        def ag_recv_desc(hf, src):
            return pltpu.make_async_remote_copy(
                src_ref=ag_ref.at[hf, 0],
                dst_ref=ag_ref.at[hf, src],
                send_sem=rs_send_sems.at[0],
                recv_sem=ag_recv_sems.at[hf * 4 + src],
                device_id=(src,),
                device_id_type=pl.DeviceIdType.MESH,
            )

        rs_order = (2, 1, 3)
        for j in range(N_DEV - 1):
            tgt = lax.rem(my + rs_order[j], N_DEV)
            chunk = compute_chunk(tgt).astype(jnp.bfloat16)
            for hf in range(NHALF):
                part_ref[hf, j] = chunk[:, hf * HALF:(hf + 1) * HALF]
            if j == 0:
                pl.semaphore_wait(barrier_sem, N_DEV - 1)
            for hf in range(NHALF):
                rs_desc(hf, j, tgt).start()
        own_ctx = compute_ctx(my)

        own_rows = pl.ds(my * R, R)
        for hf in range(NHALF):
            red = project(own_ctx, wo_bf[:, hf * HALF:(hf + 1) * HALF])
            for j in range(N_DEV - 1):
                rs_desc(hf, j, my).wait_recv()
                red = red + rs_ref[hf, j].astype(jnp.float32)
            for c in range(N_DEV):
                @pl.when(c == my)
                def _():
                    ag_ref[hf, c] = red.astype(jnp.bfloat16)
            for d in (2, 1, 3):
                ag_desc(hf, d).start()
            out_ref[own_rows, hf * HALF:(hf + 1) * HALF] = red

        for d in range(1, N_DEV):
            src = lax.rem(my + d, N_DEV)
            src_rows = pl.ds(src * R, R)
            for hf in range(NHALF):
                ag_recv_desc(hf, src).wait_recv()
                out_ref[src_rows, hf * HALF:(hf + 1) * HALF] = (
                    ag_ref[hf, src].astype(jnp.float32))

        for j in range(N_DEV - 1):
            for hf in range(NHALF):
                rs_desc(hf, j, lax.rem(my + 1 + j, N_DEV)).wait_send()
        for hf in range(NHALF):
            for d in range(1, N_DEV):
                ag_desc(hf, d).wait_send()

    out_shape = jax.ShapeDtypeStruct((B * Sq, D_MODEL), jnp.float32)
    out_flat = pl.pallas_call(
        body,
        out_shape=out_shape,
        in_specs=[pl.BlockSpec(memory_space=pltpu.VMEM)] * 5,
        out_specs=pl.BlockSpec(memory_space=pltpu.VMEM),
        scratch_shapes=[
            pltpu.VMEM((D_MODEL, H_PER * Dh), jnp.bfloat16),
            pltpu.VMEM((H_PER * Dh, D_MODEL), jnp.bfloat16),
            pltpu.VMEM((NHALF, N_DEV - 1, R, HALF), jnp.bfloat16),
            pltpu.VMEM((NHALF, N_DEV - 1, R, HALF), jnp.bfloat16),
            pltpu.VMEM((NHALF, N_DEV, R, HALF), jnp.bfloat16),
            pltpu.SemaphoreType.DMA((NHALF * (N_DEV - 1),)),
            pltpu.SemaphoreType.DMA((NHALF * (N_DEV - 1),)),
            pltpu.SemaphoreType.DMA((NHALF * (N_DEV - 1),)),
            pltpu.SemaphoreType.DMA((NHALF * N_DEV,)),
        ],
        compiler_params=pltpu.CompilerParams(collective_id=0),
    )(x_flat, Wq, K, V, Wo)
    return out_flat.reshape(B, Sq, D_MODEL)
